# baseline (speedup 1.0000x reference)
import functools

import jax
import jax.numpy as jnp
from jax import lax
from jax.experimental import pallas as pl
from jax.experimental.pallas import tpu as pltpu
from jax.experimental.pallas import tpu_sc as plsc

_BATCH = 16384
_K = 64
_N = 100000
_RANGE = 3072
_W = 768
_CH = 1024
_NRING = 16


def _nmf_body(u_hbm, i_hbm, pt_hbm, qt_hbm, rmp_hbm, rmq_hbm,
              win, win128, win32, ibu, ibi, lu, li, cbuf, stage, fired_ref,
              sem_out, sem_in):
    cid = lax.axis_index("c")
    sid = lax.axis_index("s")
    wid = sid * 2 + cid
    lanes = lax.iota(jnp.int32, 16)

    lo = wid * _RANGE
    hi = lo + _RANGE
    lo2 = jnp.where(wid == 0, 98304,
                    jnp.where(wid == 1, 99072,
                              jnp.where(wid == 2, 99840, 0)))
    hi2 = jnp.where(wid == 0, 99072,
                    jnp.where(wid == 1, 99840,
                              jnp.where(wid == 2, _N, 0)))

    fifteen = jnp.full((16,), 15, jnp.int32)

    def bcast(v, j):
        return lax.gather(
            v, j.reshape(16, 1),
            lax.GatherDimensionNumbers(
                offset_dims=(), collapsed_slice_dims=(0,),
                start_index_map=(0,)),
            (1,),
            mode=lax.GatherScatterMode.PROMISE_IN_BOUNDS,
        )

    def scan_chunk(c, counts):
        pltpu.sync_copy(u_hbm.at[pl.ds(c * _CH, _CH)], ibu)
        pltpu.sync_copy(i_hbm.at[pl.ds(c * _CH, _CH)], ibi)

        def scan_vec(g, counts):
            cnt_u, cnt_i = counts
            bvec = c * _CH + g * 16 + lanes
            vu = ibu[pl.ds(pl.multiple_of(g * 16, 16), 16)]
            vi = ibi[pl.ds(pl.multiple_of(g * 16, 16), 16)]
            mu = ((vu >= lo) & (vu < hi)) | ((vu >= lo2) & (vu < hi2))
            mi = ((vi >= lo) & (vi < hi)) | ((vi >= lo2) & (vi < hi2))
            cu = jnp.cumsum(mu.astype(jnp.int32))
            ci = jnp.cumsum(mi.astype(jnp.int32))
            plsc.store_scatter(
                lu.at[pl.ds(0, _BATCH)], [cnt_u + cu - 1],
                (vu << 14) | bvec, mask=mu)
            plsc.store_scatter(
                li.at[pl.ds(0, _BATCH)], [cnt_i + ci - 1],
                (vi << 14) | bvec, mask=mi)
            return cnt_u + bcast(cu, fifteen), cnt_i + bcast(ci, fifteen)

        return lax.fori_loop(0, _CH // 16, scan_vec, counts)

    zero16 = jnp.zeros((16,), jnp.int32)
    cnt_u_v, cnt_i_v = lax.fori_loop(
        0, _BATCH // _CH, scan_chunk, (zero16, zero16))
    cnt_u = jnp.max(cnt_u_v)
    cnt_i = jnp.max(cnt_i_v)

    def do_side(lst, cnt, rm_hbm, wref, wsize, start):
        def vec_loop(t, prev_n):
            pk = lst[pl.ds(pl.multiple_of(t * 16, 16), 16)]
            ids = pk >> 14
            m = (ids >= start) & (ids < start + wsize) \
                & (t * 16 + lanes < cnt)
            cums = jnp.cumsum(m.astype(jnp.int32))
            n = jnp.max(cums)

            def nonzero():
                plsc.store_compressed(cbuf.at[pl.ds(0, 16)], pk, mask=m)
                cb_v = cbuf[pl.ds(0, 16)]
                ul_v = (cb_v >> 14) - start
                b_v = cb_v & (_BATCH - 1)

                lax.fori_loop(
                    0, prev_n,
                    lambda _, c: (pltpu.make_async_copy(
                        stage.at[0], rm_hbm.at[0], sem_out).wait(), c)[1],
                    0)

                def match_loop(mm, carry):
                    ul = bcast(ul_v, jnp.full((16,), 0, jnp.int32) + mm)
                    b = jnp.sum(jnp.where(lanes == mm, b_v, 0))
                    for cc in range(_K // 16):
                        colv = plsc.load_gather(
                            wref, [lanes + 16 * cc, ul])
                        stage[mm, pl.ds(16 * cc, 16)] = colv
                    pltpu.make_async_copy(
                        stage.at[mm], rm_hbm.at[b], sem_out).start()
                    return carry

                lax.fori_loop(0, n, match_loop, 0)
                return n

            return lax.cond(n > 0, nonzero, lambda: prev_n)

        last_n = lax.fori_loop(0, (cnt + 15) // 16, vec_loop, 0)
        lax.fori_loop(
            0, last_n,
            lambda _, c: (pltpu.make_async_copy(
                stage.at[0], rm_hbm.at[0], sem_out).wait(), c)[1],
            0)

    def do_window(table_hbm, lst, cnt, rm_hbm, wref, wsize, start):
        pltpu.sync_copy(table_hbm.at[:, pl.ds(start, wsize)], wref)
        do_side(lst, cnt, rm_hbm, wref, wsize, start)

    for r in range(_RANGE // _W):
        start = pl.multiple_of(lo + r * _W, 128)
        do_window(pt_hbm, lu, cnt_u, rmp_hbm, win, _W, start)
        do_window(qt_hbm, li, cnt_i, rmq_hbm, win, _W, start)

    @pl.when(wid == 0)
    def _():
        do_window(pt_hbm, lu, cnt_u, rmp_hbm, win, _W, 98304)
        do_window(qt_hbm, li, cnt_i, rmq_hbm, win, _W, 98304)

    @pl.when(wid == 1)
    def _():
        do_window(pt_hbm, lu, cnt_u, rmp_hbm, win, _W, 99072)
        do_window(qt_hbm, li, cnt_i, rmq_hbm, win, _W, 99072)

    @pl.when(wid == 2)
    def _():
        do_window(pt_hbm, lu, cnt_u, rmp_hbm, win128, 128, 99840)
        do_window(qt_hbm, li, cnt_i, rmq_hbm, win128, 128, 99840)
        do_window(pt_hbm, lu, cnt_u, rmp_hbm, win32, 32, 99968)
        do_window(qt_hbm, li, cnt_i, rmq_hbm, win32, 32, 99968)


_nmf_extract = functools.partial(
    pl.kernel,
    out_type=(jax.ShapeDtypeStruct((_BATCH, _K), jnp.float32),
              jax.ShapeDtypeStruct((_BATCH, _K), jnp.float32)),
    mesh=plsc.VectorSubcoreMesh(core_axis_name="c", subcore_axis_name="s"),
    compiler_params=pltpu.CompilerParams(needs_layout_passes=False),
    scratch_types=[
        pltpu.VMEM((_K, _W), jnp.float32),
        pltpu.VMEM((_K, 128), jnp.float32),
        pltpu.VMEM((_K, 32), jnp.float32),
        pltpu.VMEM((_CH,), jnp.int32),
        pltpu.VMEM((_CH,), jnp.int32),
        pltpu.VMEM((_BATCH,), jnp.int32),
        pltpu.VMEM((_BATCH,), jnp.int32),
        pltpu.VMEM((16,), jnp.int32),
        pltpu.VMEM((_NRING, _K), jnp.float32),
        pltpu.SMEM((1,), jnp.int32),
        pltpu.SemaphoreType.DMA,
        pltpu.SemaphoreType.DMA,
    ],
)(_nmf_body)


_DOTB = 1024


def _dot_body(p_ref, q_ref, o_ref):
    o_ref[...] = jnp.sum(p_ref[...] * q_ref[...], axis=1)


_dot_tc = pl.pallas_call(
    _dot_body,
    grid=(_BATCH // _DOTB,),
    in_specs=[
        pl.BlockSpec((_DOTB, _K), lambda i: (i, 0)),
        pl.BlockSpec((_DOTB, _K), lambda i: (i, 0)),
    ],
    out_specs=pl.BlockSpec((_DOTB,), lambda i: (i,)),
    out_shape=jax.ShapeDtypeStruct((_BATCH,), jnp.float32),
)


def kernel(train_x, P, Q):
    user_id = train_x[:, 0].astype(jnp.int32)
    item_id = train_x[:, 1].astype(jnp.int32)
    rmp, rmq = _nmf_extract(user_id, item_id, P.T, Q.T)
    return _dot_tc(rmp, rmq)


# speedup vs baseline: 1.0002x; 1.0002x over previous
import functools

import jax
import jax.numpy as jnp
from jax import lax
from jax.experimental import pallas as pl
from jax.experimental.pallas import tpu as pltpu
from jax.experimental.pallas import tpu_sc as plsc

_BATCH = 16384
_K = 64
_N = 100000
_RANGE = 3072
_W = 768
_CH = 1024
_NRING = 16


def _nmf_body(u_hbm, i_hbm, pt_hbm, qt_hbm, rmp_hbm, rmq_hbm,
              win, win128, win32, ibu, ibi, lu, li, cbuf, stage, fired_ref,
              sem_out, sem_in):
    cid = lax.axis_index("c")
    sid = lax.axis_index("s")
    wid = sid * 2 + cid
    lanes = lax.iota(jnp.int32, 16)

    lo = wid * _RANGE
    hi = lo + _RANGE
    lo2 = jnp.where(wid == 0, 98304,
                    jnp.where(wid == 1, 99072,
                              jnp.where(wid == 2, 99840, 0)))
    hi2 = jnp.where(wid == 0, 99072,
                    jnp.where(wid == 1, 99840,
                              jnp.where(wid == 2, _N, 0)))

    fifteen = jnp.full((16,), 15, jnp.int32)

    def bcast(v, j):
        return lax.gather(
            v, j.reshape(16, 1),
            lax.GatherDimensionNumbers(
                offset_dims=(), collapsed_slice_dims=(0,),
                start_index_map=(0,)),
            (1,),
            mode=lax.GatherScatterMode.PROMISE_IN_BOUNDS,
        )

    def scan_chunk(c, counts):
        pltpu.sync_copy(u_hbm.at[pl.ds(c * _CH, _CH)], ibu)
        pltpu.sync_copy(i_hbm.at[pl.ds(c * _CH, _CH)], ibi)

        def scan_vec(g, counts):
            cnt_u, cnt_i = counts
            for r in range(4):
                bvec = c * _CH + g * 64 + r * 16 + lanes
                off = pl.multiple_of(g * 64 + r * 16, 16)
                vu = ibu[pl.ds(off, 16)]
                vi = ibi[pl.ds(off, 16)]
                mu = ((vu >= lo) & (vu < hi)) | ((vu >= lo2) & (vu < hi2))
                mi = ((vi >= lo) & (vi < hi)) | ((vi >= lo2) & (vi < hi2))
                cu = jnp.cumsum(mu.astype(jnp.int32))
                ci = jnp.cumsum(mi.astype(jnp.int32))
                plsc.store_scatter(
                    lu.at[pl.ds(0, _BATCH)], [cnt_u + cu - 1],
                    (vu << 14) | bvec, mask=mu)
                plsc.store_scatter(
                    li.at[pl.ds(0, _BATCH)], [cnt_i + ci - 1],
                    (vi << 14) | bvec, mask=mi)
                cnt_u = cnt_u + bcast(cu, fifteen)
                cnt_i = cnt_i + bcast(ci, fifteen)
            return cnt_u, cnt_i

        return lax.fori_loop(0, _CH // 64, scan_vec, counts)

    zero16 = jnp.zeros((16,), jnp.int32)
    cnt_u_v, cnt_i_v = lax.fori_loop(
        0, _BATCH // _CH, scan_chunk, (zero16, zero16))
    cnt_u = jnp.max(cnt_u_v)
    cnt_i = jnp.max(cnt_i_v)

    def do_side(lst, cnt, rm_hbm, wref, wsize, start):
        def vec_loop(t, prev_n):
            pk = lst[pl.ds(pl.multiple_of(t * 16, 16), 16)]
            ids = pk >> 14
            m = (ids >= start) & (ids < start + wsize) \
                & (t * 16 + lanes < cnt)
            cums = jnp.cumsum(m.astype(jnp.int32))
            n = jnp.max(cums)

            def nonzero():
                plsc.store_compressed(cbuf.at[pl.ds(0, 16)], pk, mask=m)
                cb_v = cbuf[pl.ds(0, 16)]
                ul_v = (cb_v >> 14) - start
                b_v = cb_v & (_BATCH - 1)

                lax.fori_loop(
                    0, prev_n,
                    lambda _, c: (pltpu.make_async_copy(
                        stage.at[0], rm_hbm.at[0], sem_out).wait(), c)[1],
                    0)

                def match_loop(mm, carry):
                    ul = bcast(ul_v, jnp.full((16,), 0, jnp.int32) + mm)
                    b = jnp.sum(jnp.where(lanes == mm, b_v, 0))
                    for cc in range(_K // 16):
                        colv = plsc.load_gather(
                            wref, [lanes + 16 * cc, ul])
                        stage[mm, pl.ds(16 * cc, 16)] = colv
                    pltpu.make_async_copy(
                        stage.at[mm], rm_hbm.at[b], sem_out).start()
                    return carry

                lax.fori_loop(0, n, match_loop, 0)
                return n

            return lax.cond(n > 0, nonzero, lambda: prev_n)

        last_n = lax.fori_loop(0, (cnt + 15) // 16, vec_loop, 0)
        lax.fori_loop(
            0, last_n,
            lambda _, c: (pltpu.make_async_copy(
                stage.at[0], rm_hbm.at[0], sem_out).wait(), c)[1],
            0)

    def do_window(table_hbm, lst, cnt, rm_hbm, wref, wsize, start):
        pltpu.sync_copy(table_hbm.at[:, pl.ds(start, wsize)], wref)
        do_side(lst, cnt, rm_hbm, wref, wsize, start)

    for r in range(_RANGE // _W):
        start = pl.multiple_of(lo + r * _W, 128)
        do_window(pt_hbm, lu, cnt_u, rmp_hbm, win, _W, start)
        do_window(qt_hbm, li, cnt_i, rmq_hbm, win, _W, start)

    @pl.when(wid == 0)
    def _():
        do_window(pt_hbm, lu, cnt_u, rmp_hbm, win, _W, 98304)
        do_window(qt_hbm, li, cnt_i, rmq_hbm, win, _W, 98304)

    @pl.when(wid == 1)
    def _():
        do_window(pt_hbm, lu, cnt_u, rmp_hbm, win, _W, 99072)
        do_window(qt_hbm, li, cnt_i, rmq_hbm, win, _W, 99072)

    @pl.when(wid == 2)
    def _():
        do_window(pt_hbm, lu, cnt_u, rmp_hbm, win128, 128, 99840)
        do_window(qt_hbm, li, cnt_i, rmq_hbm, win128, 128, 99840)
        do_window(pt_hbm, lu, cnt_u, rmp_hbm, win32, 32, 99968)
        do_window(qt_hbm, li, cnt_i, rmq_hbm, win32, 32, 99968)


_nmf_extract = functools.partial(
    pl.kernel,
    out_type=(jax.ShapeDtypeStruct((_BATCH, _K), jnp.float32),
              jax.ShapeDtypeStruct((_BATCH, _K), jnp.float32)),
    mesh=plsc.VectorSubcoreMesh(core_axis_name="c", subcore_axis_name="s"),
    compiler_params=pltpu.CompilerParams(needs_layout_passes=False),
    scratch_types=[
        pltpu.VMEM((_K, _W), jnp.float32),
        pltpu.VMEM((_K, 128), jnp.float32),
        pltpu.VMEM((_K, 32), jnp.float32),
        pltpu.VMEM((_CH,), jnp.int32),
        pltpu.VMEM((_CH,), jnp.int32),
        pltpu.VMEM((_BATCH,), jnp.int32),
        pltpu.VMEM((_BATCH,), jnp.int32),
        pltpu.VMEM((16,), jnp.int32),
        pltpu.VMEM((_NRING, _K), jnp.float32),
        pltpu.SMEM((1,), jnp.int32),
        pltpu.SemaphoreType.DMA,
        pltpu.SemaphoreType.DMA,
    ],
)(_nmf_body)


_DOTB = 1024


def _dot_body(p_ref, q_ref, o_ref):
    o_ref[...] = jnp.sum(p_ref[...] * q_ref[...], axis=1)


_dot_tc = pl.pallas_call(
    _dot_body,
    grid=(_BATCH // _DOTB,),
    in_specs=[
        pl.BlockSpec((_DOTB, _K), lambda i: (i, 0)),
        pl.BlockSpec((_DOTB, _K), lambda i: (i, 0)),
    ],
    out_specs=pl.BlockSpec((_DOTB,), lambda i: (i,)),
    out_shape=jax.ShapeDtypeStruct((_BATCH,), jnp.float32),
)


def kernel(train_x, P, Q):
    user_id = train_x[:, 0].astype(jnp.int32)
    item_id = train_x[:, 1].astype(jnp.int32)
    rmp, rmq = _nmf_extract(user_id, item_id, P.T, Q.T)
    return _dot_tc(rmp, rmq)


# speedup vs baseline: 1.8601x; 1.8598x over previous
import functools

import jax
import jax.numpy as jnp
from jax import lax
from jax.experimental import pallas as pl
from jax.experimental.pallas import tpu as pltpu
from jax.experimental.pallas import tpu_sc as plsc

_BATCH = 16384
_K = 64
_NUM_WORKERS = 32
_BPW = _BATCH // _NUM_WORKERS
_CHUNK = 256
_NCHUNKS = _BPW // _CHUNK


def _nmf_body(u_hbm, i_hbm, p_hbm, q_hbm, out_hbm,
              idx_u, idx_i, rows_u, rows_i, out_v, sem_u, sem_i):
    cid = lax.axis_index("c")
    sid = lax.axis_index("s")
    wid = sid * 2 + cid
    base = pl.multiple_of(wid * _BPW, _BPW)

    pltpu.sync_copy(u_hbm.at[pl.ds(base, _BPW)], idx_u)
    pltpu.sync_copy(i_hbm.at[pl.ds(base, _BPW)], idx_i)

    lanes = lax.iota(jnp.int32, 16)

    for chunk in range(_NCHUNKS):
        off = chunk * _CHUNK

        def fire(g, carry, _off=off):
            s = pl.multiple_of(_off + g * 16, 16)
            vu = idx_u[pl.ds(s, 16)]
            vi = idx_i[pl.ds(s, 16)]
            for j in range(16):
                ru = jnp.sum(jnp.where(lanes == j, vu, 0))
                ri = jnp.sum(jnp.where(lanes == j, vi, 0))
                b = g * 16 + j
                pltpu.make_async_copy(
                    p_hbm.at[ru], rows_u.at[b], sem_u).start()
                pltpu.make_async_copy(
                    q_hbm.at[ri], rows_i.at[b], sem_i).start()
            return carry

        lax.fori_loop(0, _CHUNK // 16, fire, 0)

        def drain(b, carry):
            pltpu.make_async_copy(
                p_hbm.at[0], rows_u.at[0], sem_u).wait()
            pltpu.make_async_copy(
                q_hbm.at[0], rows_i.at[0], sem_i).wait()
            return carry

        lax.fori_loop(0, _CHUNK, drain, 0)

        def grp(g, carry, _off=off):
            row = g * 16 + lanes
            acc = jnp.zeros((16,), jnp.float32)
            for k in range(_K):
                col = (lanes + k) & (_K - 1)
                acc = acc + (plsc.load_gather(rows_u, [row, col])
                             * plsc.load_gather(rows_i, [row, col]))
            out_v[pl.ds(pl.multiple_of(_off + g * 16, 16), 16)] = acc
            return carry

        lax.fori_loop(0, _CHUNK // 16, grp, 0)

    pltpu.sync_copy(out_v, out_hbm.at[pl.ds(base, _BPW)])


_nmf_sc = functools.partial(
    pl.kernel,
    out_type=jax.ShapeDtypeStruct((_BATCH,), jnp.float32),
    mesh=plsc.VectorSubcoreMesh(core_axis_name="c", subcore_axis_name="s"),
    compiler_params=pltpu.CompilerParams(needs_layout_passes=False),
    scratch_types=[
        pltpu.VMEM((_BPW,), jnp.int32),
        pltpu.VMEM((_BPW,), jnp.int32),
        pltpu.VMEM((_CHUNK, _K), jnp.float32),
        pltpu.VMEM((_CHUNK, _K), jnp.float32),
        pltpu.VMEM((_BPW,), jnp.float32),
        pltpu.SemaphoreType.DMA,
        pltpu.SemaphoreType.DMA,
    ],
)(_nmf_body)


def kernel(train_x, P, Q):
    user_id = train_x[:, 0].astype(jnp.int32)
    item_id = train_x[:, 1].astype(jnp.int32)
    return _nmf_sc(user_id, item_id, P, Q)
